# baseline (speedup 1.0000x reference)
import functools

import jax
import jax.numpy as jnp
from jax import lax
from jax.experimental import pallas as pl
from jax.experimental.pallas import tpu as pltpu
from jax.experimental.pallas import tpu_sc as plsc

K = 32
F = 128
NC = 2
NS = 16
NW = NC * NS
CH = 40
MININT = -2147483648


NB = 5


def _make_sc_agg(n2, e):
    ew = e // NW
    nchunks = ew // CH
    nrounds = nchunks // NB
    rows_per_tile = n2 // NS
    zrows = 32
    nzcopy = rows_per_tile // zrows

    mesh = plsc.VectorSubcoreMesh(core_axis_name="c", subcore_axis_name="s")

    @functools.partial(
        pl.kernel,
        mesh=mesh,
        compiler_params=pltpu.CompilerParams(needs_layout_passes=False),
        out_type=(
            jax.ShapeDtypeStruct((NC, n2, F), jnp.float32),
            jax.ShapeDtypeStruct((NW, n2), jnp.float32),
        ),
        scratch_types=[
            [pltpu.VMEM((CH,), jnp.int32) for _ in range(NB)],
            [pltpu.VMEM((CH,), jnp.int32) for _ in range(NB)],
            [pltpu.VMEM((CH, F), jnp.float32) for _ in range(NB)],
            pltpu.VMEM((zrows, F), jnp.float32),
            pltpu.VMEM((n2,), jnp.float32),
            pltpu.VMEM_SHARED((n2, F), jnp.float32),
            [pltpu.SemaphoreType.DMA for _ in range(NB)],
            [pltpu.SemaphoreType.DMA for _ in range(NB)],
        ],
    )
    def agg(h_hbm, src_hbm, dst_hbm, p_hbm, deg_hbm,
            idx_d, idx_s, rows, zbuf, deg_loc, acc, gsem, ssem):
        c = lax.axis_index("c")
        s = lax.axis_index("s")
        wid = s * NC + c

        zero16 = jnp.zeros((16,), jnp.float32)

        def zb_body(t, _):
            zbuf[t // 8, pl.ds((t % 8) * 16, 16)] = zero16
            return _
        lax.fori_loop(0, zrows * 8, zb_body, 0)

        def zd_body(t, _):
            deg_loc[pl.ds(t * 16, 16)] = zero16
            return _
        lax.fori_loop(0, n2 // 16, zd_body, 0)

        row0 = s * rows_per_tile
        def zc_body(t, _):
            pltpu.sync_copy(zbuf, acc.at[pl.ds(row0 + t * zrows, zrows)])
            return _
        lax.fori_loop(0, nzcopy, zc_body, 0)
        plsc.subcore_barrier()

        ebase = wid * ew

        def load_and_gather(b, j):
            base = ebase + j * CH
            pltpu.sync_copy(dst_hbm.at[pl.ds(base, CH)], idx_d[b])
            pltpu.sync_copy(src_hbm.at[pl.ds(base, CH)], idx_s[b])
            pltpu.async_copy(h_hbm.at[idx_d[b]], rows[b], gsem[b])

        def wait_gather(b):
            pltpu.make_async_copy(h_hbm.at[idx_d[b]], rows[b], gsem[b]).wait()

        def scatter(b):
            pltpu.async_copy(rows[b], acc.at[idx_s[b]], ssem[b], add=True)

        def wait_scatter(b):
            pltpu.make_async_copy(rows[b], acc.at[idx_s[b]], ssem[b]).wait()

        ones16 = jnp.ones((16,), jnp.float32)

        def deg_update(b):
            for g in range(CH // 16):
                iv = idx_s[b][pl.ds(g * 16, 16)]
                plsc.addupdate_scatter(deg_loc, [iv], ones16)
            if CH % 16:
                iv = idx_s[b][pl.ds(CH - 16, 16)]
                m = lax.iota(jnp.int32, 16) >= (16 - (CH % 16))
                plsc.addupdate_scatter(deg_loc, [iv], ones16, mask=m)

        def round_body(r, carry):
            for b in range(NB):
                @pl.when(r > 0)
                def _drain(b=b):
                    wait_scatter(b)
                load_and_gather(b, r * NB + b)
            for b in range(NB):
                wait_gather(b)
                scatter(b)
                deg_update(b)
            return carry
        lax.fori_loop(0, nrounds, round_body, 0)
        for b in range(NB):
            wait_scatter(b)
        plsc.subcore_barrier()

        pltpu.sync_copy(acc.at[pl.ds(row0, rows_per_tile)],
                        p_hbm.at[c, pl.ds(row0, rows_per_tile)])
        pltpu.sync_copy(deg_loc, deg_hbm.at[wid])

    return agg


def _maxk_tc(h):
    b = lax.bitcast_convert_type(h, jnp.int32)
    ki = jnp.where(b >= 0, b, ~(b ^ jnp.int32(MININT)))
    ones_col = jnp.ones((F, 1), jnp.float32)
    kf = jnp.float32(K)

    def cnt_ge(cand):
        m = jnp.where(ki >= cand, jnp.float32(1.0), jnp.float32(0.0))
        return lax.dot_general(m, ones_col, (((1,), (0,)), ((), ())),
                               preferred_element_type=jnp.float32)

    t = jnp.where(cnt_ge(jnp.int32(0)) >= kf, jnp.int32(0),
                  jnp.int32(MININT))
    for bit in range(30, -1, -1):
        cand = t + jnp.int32(1 << bit)
        t = jnp.where(cnt_ge(cand) >= kf, cand, t)
    return jnp.where(ki >= t, h, jnp.float32(0.0))


def _dotT(a, w):
    return lax.dot_general(a, w, (((1,), (1,)), ((), ())),
                           preferred_element_type=jnp.float32)


def _in_body(x_ref, w_ref, b_ref, o_ref):
    h = _dotT(x_ref[...], w_ref[...]) + b_ref[...]
    o_ref[...] = _maxk_tc(h)


def _neigh_block(p_ref, degp, wn, bn):
    deg = jnp.sum(degp, axis=0, keepdims=True)
    eye = (lax.broadcasted_iota(jnp.int32, (F, F), 0)
           == lax.broadcasted_iota(jnp.int32, (F, F), 1)).astype(jnp.float32)
    ones_ff = jnp.ones((F, F), jnp.float32)
    p = p_ref[0] + p_ref[1]
    groups = []
    for u in range(bn // F):
        degrow = deg[:, u * F:(u + 1) * F]
        degcol = lax.dot_general(eye * degrow, ones_ff,
                                 (((1,), (0,)), ((), ())),
                                 preferred_element_type=jnp.float32)
        groups.append(p[u * F:(u + 1) * F, :] / (degcol + 1e-6))
    aggn = jnp.concatenate(groups, axis=0) if len(groups) > 1 else groups[0]
    return _dotT(aggn, wn)


def _layer_body(hm_ref, p_ref, degp_ref, ws_ref, wn_ref, o_ref, *, bn):
    h = _dotT(hm_ref[...], ws_ref[...]) + _neigh_block(
        p_ref, degp_ref[...], wn_ref[...], bn)
    o_ref[...] = _maxk_tc(h)


def _final_body(hm_ref, p_ref, degp_ref, ws_ref, wn_ref,
                wo_ref, bo_ref, o_ref, *, bn):
    h = _dotT(hm_ref[...], ws_ref[...]) + _neigh_block(
        p_ref, degp_ref[...], wn_ref[...], bn)
    o_ref[...] = _dotT(h, wo_ref[...]) + bo_ref[...]


def _full(shape):
    return pl.BlockSpec(shape, lambda i: tuple(0 for _ in shape))


def _mm_in(x, w, b, n2, bn):
    return pl.pallas_call(
        _in_body,
        grid=(n2 // bn,),
        in_specs=[
            pl.BlockSpec((bn, F), lambda i: (i, 0)),
            _full((F, F)),
            _full((1, F)),
        ],
        out_specs=pl.BlockSpec((bn, F), lambda i: (i, 0)),
        out_shape=jax.ShapeDtypeStruct((n2, F), jnp.float32),
    )(x, w, b)


def _mm_layer(hm, p, degp, ws, wn, n2, bn):
    return pl.pallas_call(
        functools.partial(_layer_body, bn=bn),
        grid=(n2 // bn,),
        in_specs=[
            pl.BlockSpec((bn, F), lambda i: (i, 0)),
            pl.BlockSpec((2, bn, F), lambda i: (0, i, 0)),
            pl.BlockSpec((NW, bn), lambda i: (0, i)),
            _full((F, F)),
            _full((F, F)),
        ],
        out_specs=pl.BlockSpec((bn, F), lambda i: (i, 0)),
        out_shape=jax.ShapeDtypeStruct((n2, F), jnp.float32),
    )(hm, p, degp, ws, wn)


def _mm_final(hm, p, degp, ws, wn, wo, bo, n2, bn):
    return pl.pallas_call(
        functools.partial(_final_body, bn=bn),
        grid=(n2 // bn,),
        in_specs=[
            pl.BlockSpec((bn, F), lambda i: (i, 0)),
            pl.BlockSpec((2, bn, F), lambda i: (0, i, 0)),
            pl.BlockSpec((NW, bn), lambda i: (0, i)),
            _full((F, F)),
            _full((F, F)),
            _full((F, F)),
            _full((1, F)),
        ],
        out_specs=pl.BlockSpec((bn, F), lambda i: (i, 0)),
        out_shape=jax.ShapeDtypeStruct((n2, F), jnp.float32),
    )(hm, p, degp, ws, wn, wo, bo)


def kernel(x, edge_index, W_in, b_in, W_self_0, W_neigh_0,
           W_self_1, W_neigh_1, W_self_2, W_neigh_2, W_out, b_out):
    n = x.shape[0]
    e = edge_index.shape[1]
    n2 = ((n + 1023) // 1024) * 1024
    assert e % (NW * CH * NB) == 0 and n2 % (NS * 128) == 0

    src = edge_index[0].astype(jnp.int32)
    dst = edge_index[1].astype(jnp.int32)
    x2 = jnp.pad(x, ((0, n2 - n), (0, 0)))

    sc_agg = _make_sc_agg(n2, e)

    hm = _mm_in(x2, W_in, b_in.reshape(1, F), n2, 1024)
    for i, (ws, wn) in enumerate([(W_self_0, W_neigh_0),
                                  (W_self_1, W_neigh_1),
                                  (W_self_2, W_neigh_2)]):
        p, degp = sc_agg(hm, src, dst)
        if i < 2:
            hm = _mm_layer(hm, p, degp, ws, wn, n2, 1024)
        else:
            out = _mm_final(hm, p, degp, ws, wn,
                            W_out, b_out.reshape(1, F), n2, 1024)
    return out[:n]


# speedup vs baseline: 1.0498x; 1.0498x over previous
import functools

import jax
import jax.numpy as jnp
from jax import lax
from jax.experimental import pallas as pl
from jax.experimental.pallas import tpu as pltpu
from jax.experimental.pallas import tpu_sc as plsc

K = 32
F = 128
NC = 2
NS = 16
NW = NC * NS
CH = 32
MININT = -2147483648


NB = 4


def _make_sc_agg(n2, ew):
    nchunks = ew // CH
    nrounds = nchunks // NB
    rows_per_tile = n2 // NS
    nzcopy = rows_per_tile // CH

    mesh = plsc.VectorSubcoreMesh(core_axis_name="c", subcore_axis_name="s")

    @functools.partial(
        pl.kernel,
        mesh=mesh,
        compiler_params=pltpu.CompilerParams(needs_layout_passes=False),
        out_type=(
            jax.ShapeDtypeStruct((NC, n2, F), jnp.float32),
            jax.ShapeDtypeStruct((NW, n2), jnp.float32),
        ),
        scratch_types=[
            pltpu.VMEM((nchunks * CH,), jnp.int32),
            pltpu.VMEM((nchunks * CH,), jnp.int32),
            [pltpu.VMEM((CH,), jnp.int32) for _ in range(NB)],
            [pltpu.VMEM((CH, F), jnp.float32) for _ in range(NB)],
            pltpu.VMEM((n2,), jnp.float32),
            pltpu.VMEM_SHARED((n2, F), jnp.float32),
            [pltpu.SemaphoreType.DMA for _ in range(NB)],
            [pltpu.SemaphoreType.DMA for _ in range(NB)],
            pltpu.SemaphoreType.DMA,
        ],
    )
    def agg(h_hbm, src_hbm, dst_hbm, p_hbm, deg_hbm,
            idx_d, idx_s, idx_sb, rows, deg_loc, acc, gsem, ssem, zsem):
        c = lax.axis_index("c")
        s = lax.axis_index("s")
        wid = s * NC + c
        row0 = s * rows_per_tile

        pltpu.async_copy(dst_hbm.at[pl.ds(wid * nchunks * CH, nchunks * CH)],
                         idx_d, zsem)
        pltpu.async_copy(src_hbm.at[pl.ds(wid * nchunks * CH, nchunks * CH)],
                         idx_s, zsem)

        zero16 = jnp.zeros((16,), jnp.float32)

        def zb_body(t, _):
            rows[0][t // 8, pl.ds((t % 8) * 16, 16)] = zero16
            return _
        lax.fori_loop(0, CH * 8, zb_body, 0)

        def zc_body(t, _):
            pltpu.async_copy(rows[0], acc.at[pl.ds(row0 + t * CH, CH)],
                             gsem[0])
            return _
        lax.fori_loop(0, nzcopy, zc_body, 0)

        def zd_body(t, _):
            deg_loc[pl.ds(t * 16, 16)] = zero16
            return _
        lax.fori_loop(0, n2 // 16, zd_body, 0)

        def zc_drain(t, _):
            pltpu.make_async_copy(rows[0], acc.at[pl.ds(row0, CH)],
                                  gsem[0]).wait()
            return _
        lax.fori_loop(0, nzcopy, zc_drain, 0)
        pltpu.make_async_copy(dst_hbm.at[pl.ds(0, nchunks * CH)], idx_d,
                              zsem).wait()
        pltpu.make_async_copy(src_hbm.at[pl.ds(0, nchunks * CH)], idx_s,
                              zsem).wait()
        plsc.subcore_barrier()

        def gather(b, j):
            pltpu.async_copy(h_hbm.at[idx_d.at[pl.ds(j * CH, CH)]], rows[b],
                             gsem[b])

        def wait_gather(b, j):
            pltpu.make_async_copy(h_hbm.at[idx_d.at[pl.ds(j * CH, CH)]],
                                  rows[b], gsem[b]).wait()

        def scatter(b, j):
            for g in range(CH // 16):
                idx_sb[b][pl.ds(g * 16, 16)] = idx_s[pl.ds(j * CH + g * 16, 16)]
            pltpu.async_copy(rows[b], acc.at[idx_sb[b]], ssem[b], add=True)

        def wait_scatter(b, j):
            pltpu.make_async_copy(rows[b], acc.at[idx_sb[b]],
                                  ssem[b]).wait()

        ones16 = jnp.ones((16,), jnp.float32)

        def deg_update(j):
            for g in range(CH // 16):
                iv = idx_s[pl.ds(j * CH + g * 16, 16)]
                plsc.addupdate_scatter(deg_loc, [iv], ones16)

        def round_body(r, carry):
            for b in range(NB):
                @pl.when(r > 0)
                def _drain(b=b, r=r):
                    wait_scatter(b, r * NB + b - NB)
                gather(b, r * NB + b)
            for b in range(NB):
                j = r * NB + b
                wait_gather(b, j)
                scatter(b, j)
                deg_update(j)
            return carry
        lax.fori_loop(0, nrounds, round_body, 0)
        for b in range(NB):
            wait_scatter(b, (nrounds - 1) * NB + b)
        plsc.subcore_barrier()

        pltpu.sync_copy(acc.at[pl.ds(row0, rows_per_tile)],
                        p_hbm.at[c, pl.ds(row0, rows_per_tile)])
        pltpu.sync_copy(deg_loc, deg_hbm.at[wid])

    return agg


def _maxk_tc(h):
    b = lax.bitcast_convert_type(h, jnp.int32)
    ki = jnp.where(b >= 0, b, ~(b ^ jnp.int32(MININT)))
    ones_col = jnp.ones((F, 1), jnp.float32)
    kf = jnp.float32(K)

    def cnt_ge(cand):
        m = jnp.where(ki >= cand, jnp.float32(1.0), jnp.float32(0.0))
        return lax.dot_general(m, ones_col, (((1,), (0,)), ((), ())),
                               preferred_element_type=jnp.float32)

    t = jnp.where(cnt_ge(jnp.int32(0)) >= kf, jnp.int32(0),
                  jnp.int32(MININT))
    for bit in range(30, -1, -1):
        cand = t + jnp.int32(1 << bit)
        t = jnp.where(cnt_ge(cand) >= kf, cand, t)
    return jnp.where(ki >= t, h, jnp.float32(0.0))


def _dotT(a, w):
    return lax.dot_general(a, w, (((1,), (1,)), ((), ())),
                           preferred_element_type=jnp.float32)


def _in_body(x_ref, w_ref, b_ref, o_ref):
    h = _dotT(x_ref[...], w_ref[...]) + b_ref[...]
    o_ref[...] = _maxk_tc(h)


def _neigh_block(p_ref, degp, wn, bn):
    deg = jnp.sum(degp, axis=0, keepdims=True)
    eye = (lax.broadcasted_iota(jnp.int32, (F, F), 0)
           == lax.broadcasted_iota(jnp.int32, (F, F), 1)).astype(jnp.float32)
    ones_ff = jnp.ones((F, F), jnp.float32)
    p = p_ref[0] + p_ref[1]
    groups = []
    for u in range(bn // F):
        degrow = deg[:, u * F:(u + 1) * F]
        degcol = lax.dot_general(eye * degrow, ones_ff,
                                 (((1,), (0,)), ((), ())),
                                 preferred_element_type=jnp.float32)
        groups.append(p[u * F:(u + 1) * F, :] / (degcol + 1e-6))
    aggn = jnp.concatenate(groups, axis=0) if len(groups) > 1 else groups[0]
    return _dotT(aggn, wn)


def _layer_body(hm_ref, p_ref, degp_ref, ws_ref, wn_ref, o_ref, *, bn):
    h = _dotT(hm_ref[...], ws_ref[...]) + _neigh_block(
        p_ref, degp_ref[...], wn_ref[...], bn)
    o_ref[...] = _maxk_tc(h)


def _final_body(hm_ref, p_ref, degp_ref, ws_ref, wn_ref,
                wo_ref, bo_ref, o_ref, *, bn):
    h = _dotT(hm_ref[...], ws_ref[...]) + _neigh_block(
        p_ref, degp_ref[...], wn_ref[...], bn)
    o_ref[...] = _dotT(h, wo_ref[...]) + bo_ref[...]


def _full(shape):
    return pl.BlockSpec(shape, lambda i: tuple(0 for _ in shape))


def _mm_in(x, w, b, n2, bn):
    return pl.pallas_call(
        _in_body,
        grid=(n2 // bn,),
        in_specs=[
            pl.BlockSpec((bn, F), lambda i: (i, 0)),
            _full((F, F)),
            _full((1, F)),
        ],
        out_specs=pl.BlockSpec((bn, F), lambda i: (i, 0)),
        out_shape=jax.ShapeDtypeStruct((n2, F), jnp.float32),
    )(x, w, b)


def _mm_layer(hm, p, degp, ws, wn, n2, bn):
    return pl.pallas_call(
        functools.partial(_layer_body, bn=bn),
        grid=(n2 // bn,),
        in_specs=[
            pl.BlockSpec((bn, F), lambda i: (i, 0)),
            pl.BlockSpec((2, bn, F), lambda i: (0, i, 0)),
            pl.BlockSpec((NW, bn), lambda i: (0, i)),
            _full((F, F)),
            _full((F, F)),
        ],
        out_specs=pl.BlockSpec((bn, F), lambda i: (i, 0)),
        out_shape=jax.ShapeDtypeStruct((n2, F), jnp.float32),
    )(hm, p, degp, ws, wn)


def _mm_final(hm, p, degp, ws, wn, wo, bo, n2, bn):
    return pl.pallas_call(
        functools.partial(_final_body, bn=bn),
        grid=(n2 // bn,),
        in_specs=[
            pl.BlockSpec((bn, F), lambda i: (i, 0)),
            pl.BlockSpec((2, bn, F), lambda i: (0, i, 0)),
            pl.BlockSpec((NW, bn), lambda i: (0, i)),
            _full((F, F)),
            _full((F, F)),
            _full((F, F)),
            _full((1, F)),
        ],
        out_specs=pl.BlockSpec((bn, F), lambda i: (i, 0)),
        out_shape=jax.ShapeDtypeStruct((n2, F), jnp.float32),
    )(hm, p, degp, ws, wn, wo, bo)


def kernel(x, edge_index, W_in, b_in, W_self_0, W_neigh_0,
           W_self_1, W_neigh_1, W_self_2, W_neigh_2, W_out, b_out):
    n = x.shape[0]
    e = edge_index.shape[1]
    n2 = ((n + 1023) // 1024) * 1024
    assert e % NW == 0 and n2 % (NS * CH) == 0

    ew = e // NW
    ew2 = -(-ew // (CH * NB)) * (CH * NB)
    nchunks = ew2 // CH
    src = edge_index[0].astype(jnp.int32).reshape(NW, ew)
    dst = edge_index[1].astype(jnp.int32).reshape(NW, ew)
    src = jnp.pad(src, ((0, 0), (0, ew2 - ew)), constant_values=n2 - 1)
    dst = jnp.pad(dst, ((0, 0), (0, ew2 - ew)))
    src = src.reshape(NW * ew2)
    dst = dst.reshape(NW * ew2)
    x2 = jnp.pad(x, ((0, n2 - n), (0, 0)))

    sc_agg = _make_sc_agg(n2, ew2)

    hm = _mm_in(x2, W_in, b_in.reshape(1, F), n2, 1024)
    for i, (ws, wn) in enumerate([(W_self_0, W_neigh_0),
                                  (W_self_1, W_neigh_1),
                                  (W_self_2, W_neigh_2)]):
        p, degp = sc_agg(hm, src, dst)
        if i < 2:
            hm = _mm_layer(hm, p, degp, ws, wn, n2, 1024)
        else:
            out = _mm_final(hm, p, degp, ws, wn,
                            W_out, b_out.reshape(1, F), n2, 1024)
    return out[:n]


# speedup vs baseline: 1.0566x; 1.0064x over previous
import functools

import jax
import jax.numpy as jnp
from jax import lax
from jax.experimental import pallas as pl
from jax.experimental.pallas import tpu as pltpu
from jax.experimental.pallas import tpu_sc as plsc

K = 32
F = 128
NC = 2
NS = 16
NW = NC * NS
CH = 16
MININT = -2147483648


NB = 8


def _make_sc_agg(n2, ew):
    nchunks = ew // CH
    nrounds = nchunks // NB
    rows_per_tile = n2 // NS
    nzcopy = rows_per_tile // CH

    mesh = plsc.VectorSubcoreMesh(core_axis_name="c", subcore_axis_name="s")

    @functools.partial(
        pl.kernel,
        mesh=mesh,
        compiler_params=pltpu.CompilerParams(needs_layout_passes=False),
        out_type=(
            jax.ShapeDtypeStruct((NC, n2, F), jnp.float32),
            jax.ShapeDtypeStruct((NW, n2), jnp.float32),
        ),
        scratch_types=[
            pltpu.VMEM((nchunks * CH,), jnp.int32),
            pltpu.VMEM((nchunks * CH,), jnp.int32),
            [pltpu.VMEM((CH,), jnp.int32) for _ in range(NB)],
            [pltpu.VMEM((CH, F), jnp.float32) for _ in range(NB)],
            pltpu.VMEM((n2,), jnp.float32),
            pltpu.VMEM_SHARED((n2, F), jnp.float32),
            [pltpu.SemaphoreType.DMA for _ in range(NB)],
            [pltpu.SemaphoreType.DMA for _ in range(NB)],
            pltpu.SemaphoreType.DMA,
        ],
    )
    def agg(h_hbm, src_hbm, dst_hbm, p_hbm, deg_hbm,
            idx_d, idx_s, idx_sb, rows, deg_loc, acc, gsem, ssem, zsem):
        c = lax.axis_index("c")
        s = lax.axis_index("s")
        wid = s * NC + c
        row0 = s * rows_per_tile

        pltpu.async_copy(dst_hbm.at[pl.ds(wid * nchunks * CH, nchunks * CH)],
                         idx_d, zsem)
        pltpu.async_copy(src_hbm.at[pl.ds(wid * nchunks * CH, nchunks * CH)],
                         idx_s, zsem)

        zero16 = jnp.zeros((16,), jnp.float32)

        def zb_body(t, _):
            rows[0][t // 8, pl.ds((t % 8) * 16, 16)] = zero16
            return _
        lax.fori_loop(0, CH * 8, zb_body, 0)

        def zc_body(t, _):
            pltpu.async_copy(rows[0], acc.at[pl.ds(row0 + t * CH, CH)],
                             gsem[0])
            return _
        lax.fori_loop(0, nzcopy, zc_body, 0)

        def zd_body(t, _):
            deg_loc[pl.ds(t * 16, 16)] = zero16
            return _
        lax.fori_loop(0, n2 // 16, zd_body, 0)

        def zc_drain(t, _):
            pltpu.make_async_copy(rows[0], acc.at[pl.ds(row0, CH)],
                                  gsem[0]).wait()
            return _
        lax.fori_loop(0, nzcopy, zc_drain, 0)
        pltpu.make_async_copy(dst_hbm.at[pl.ds(0, nchunks * CH)], idx_d,
                              zsem).wait()
        pltpu.make_async_copy(src_hbm.at[pl.ds(0, nchunks * CH)], idx_s,
                              zsem).wait()
        plsc.subcore_barrier()

        def gather(b, j):
            pltpu.async_copy(h_hbm.at[idx_d.at[pl.ds(j * CH, CH)]], rows[b],
                             gsem[b])

        def wait_gather(b, j):
            pltpu.make_async_copy(h_hbm.at[idx_d.at[pl.ds(j * CH, CH)]],
                                  rows[b], gsem[b]).wait()

        def scatter(b, j):
            for g in range(CH // 16):
                idx_sb[b][pl.ds(g * 16, 16)] = idx_s[pl.ds(j * CH + g * 16, 16)]
            pltpu.async_copy(rows[b], acc.at[idx_sb[b]], ssem[b], add=True)

        def wait_scatter(b, j):
            pltpu.make_async_copy(rows[b], acc.at[idx_sb[b]],
                                  ssem[b]).wait()

        ones16 = jnp.ones((16,), jnp.float32)

        def deg_update(j):
            for g in range(CH // 16):
                iv = idx_s[pl.ds(j * CH + g * 16, 16)]
                plsc.addupdate_scatter(deg_loc, [iv], ones16)

        def round_body(r, carry):
            for b in range(NB):
                @pl.when(r > 0)
                def _drain(b=b, r=r):
                    wait_scatter(b, r * NB + b - NB)
                gather(b, r * NB + b)
            for b in range(NB):
                j = r * NB + b
                wait_gather(b, j)
                scatter(b, j)
                deg_update(j)
            return carry
        lax.fori_loop(0, nrounds, round_body, 0)
        for b in range(NB):
            wait_scatter(b, (nrounds - 1) * NB + b)
        plsc.subcore_barrier()

        pltpu.sync_copy(acc.at[pl.ds(row0, rows_per_tile)],
                        p_hbm.at[c, pl.ds(row0, rows_per_tile)])
        pltpu.sync_copy(deg_loc, deg_hbm.at[wid])

    return agg


def _maxk_tc(h):
    b = lax.bitcast_convert_type(h, jnp.int32)
    ki = jnp.where(b >= 0, b, ~(b ^ jnp.int32(MININT)))
    ones_col = jnp.ones((F, 1), jnp.float32)
    kf = jnp.float32(K)

    def cnt_ge(cand):
        m = jnp.where(ki >= cand, jnp.float32(1.0), jnp.float32(0.0))
        return lax.dot_general(m, ones_col, (((1,), (0,)), ((), ())),
                               preferred_element_type=jnp.float32)

    t = jnp.where(cnt_ge(jnp.int32(0)) >= kf, jnp.int32(0),
                  jnp.int32(MININT))
    for bit in range(30, -1, -1):
        cand = t + jnp.int32(1 << bit)
        t = jnp.where(cnt_ge(cand) >= kf, cand, t)
    return jnp.where(ki >= t, h, jnp.float32(0.0))


def _dotT(a, w):
    return lax.dot_general(a, w, (((1,), (1,)), ((), ())),
                           preferred_element_type=jnp.float32)


def _in_body(x_ref, w_ref, b_ref, o_ref):
    h = _dotT(x_ref[...], w_ref[...]) + b_ref[...]
    o_ref[...] = _maxk_tc(h)


def _neigh_block(p_ref, degp, wn, bn):
    deg = jnp.sum(degp, axis=0, keepdims=True)
    eye = (lax.broadcasted_iota(jnp.int32, (F, F), 0)
           == lax.broadcasted_iota(jnp.int32, (F, F), 1)).astype(jnp.float32)
    ones_ff = jnp.ones((F, F), jnp.float32)
    p = p_ref[0] + p_ref[1]
    groups = []
    for u in range(bn // F):
        degrow = deg[:, u * F:(u + 1) * F]
        degcol = lax.dot_general(eye * degrow, ones_ff,
                                 (((1,), (0,)), ((), ())),
                                 preferred_element_type=jnp.float32)
        groups.append(p[u * F:(u + 1) * F, :] / (degcol + 1e-6))
    aggn = jnp.concatenate(groups, axis=0) if len(groups) > 1 else groups[0]
    return _dotT(aggn, wn)


def _layer_body(hm_ref, p_ref, degp_ref, ws_ref, wn_ref, o_ref, *, bn):
    h = _dotT(hm_ref[...], ws_ref[...]) + _neigh_block(
        p_ref, degp_ref[...], wn_ref[...], bn)
    o_ref[...] = _maxk_tc(h)


def _final_body(hm_ref, p_ref, degp_ref, ws_ref, wn_ref,
                wo_ref, bo_ref, o_ref, *, bn):
    h = _dotT(hm_ref[...], ws_ref[...]) + _neigh_block(
        p_ref, degp_ref[...], wn_ref[...], bn)
    o_ref[...] = _dotT(h, wo_ref[...]) + bo_ref[...]


def _full(shape):
    return pl.BlockSpec(shape, lambda i: tuple(0 for _ in shape))


def _mm_in(x, w, b, n2, bn):
    return pl.pallas_call(
        _in_body,
        grid=(n2 // bn,),
        in_specs=[
            pl.BlockSpec((bn, F), lambda i: (i, 0)),
            _full((F, F)),
            _full((1, F)),
        ],
        out_specs=pl.BlockSpec((bn, F), lambda i: (i, 0)),
        out_shape=jax.ShapeDtypeStruct((n2, F), jnp.float32),
    )(x, w, b)


def _mm_layer(hm, p, degp, ws, wn, n2, bn):
    return pl.pallas_call(
        functools.partial(_layer_body, bn=bn),
        grid=(n2 // bn,),
        in_specs=[
            pl.BlockSpec((bn, F), lambda i: (i, 0)),
            pl.BlockSpec((2, bn, F), lambda i: (0, i, 0)),
            pl.BlockSpec((NW, bn), lambda i: (0, i)),
            _full((F, F)),
            _full((F, F)),
        ],
        out_specs=pl.BlockSpec((bn, F), lambda i: (i, 0)),
        out_shape=jax.ShapeDtypeStruct((n2, F), jnp.float32),
    )(hm, p, degp, ws, wn)


def _mm_final(hm, p, degp, ws, wn, wo, bo, n2, bn):
    return pl.pallas_call(
        functools.partial(_final_body, bn=bn),
        grid=(n2 // bn,),
        in_specs=[
            pl.BlockSpec((bn, F), lambda i: (i, 0)),
            pl.BlockSpec((2, bn, F), lambda i: (0, i, 0)),
            pl.BlockSpec((NW, bn), lambda i: (0, i)),
            _full((F, F)),
            _full((F, F)),
            _full((F, F)),
            _full((1, F)),
        ],
        out_specs=pl.BlockSpec((bn, F), lambda i: (i, 0)),
        out_shape=jax.ShapeDtypeStruct((n2, F), jnp.float32),
    )(hm, p, degp, ws, wn, wo, bo)


def kernel(x, edge_index, W_in, b_in, W_self_0, W_neigh_0,
           W_self_1, W_neigh_1, W_self_2, W_neigh_2, W_out, b_out):
    n = x.shape[0]
    e = edge_index.shape[1]
    n2 = ((n + 1023) // 1024) * 1024
    assert e % NW == 0 and n2 % (NS * CH) == 0

    ew = e // NW
    ew2 = -(-ew // (CH * NB)) * (CH * NB)
    nchunks = ew2 // CH
    src = edge_index[0].astype(jnp.int32).reshape(NW, ew)
    dst = edge_index[1].astype(jnp.int32).reshape(NW, ew)
    src = jnp.pad(src, ((0, 0), (0, ew2 - ew)), constant_values=n2 - 1)
    dst = jnp.pad(dst, ((0, 0), (0, ew2 - ew)))
    src = src.reshape(NW * ew2)
    dst = dst.reshape(NW * ew2)
    x2 = jnp.pad(x, ((0, n2 - n), (0, 0)))

    sc_agg = _make_sc_agg(n2, ew2)

    hm = _mm_in(x2, W_in, b_in.reshape(1, F), n2, 1024)
    for i, (ws, wn) in enumerate([(W_self_0, W_neigh_0),
                                  (W_self_1, W_neigh_1),
                                  (W_self_2, W_neigh_2)]):
        p, degp = sc_agg(hm, src, dst)
        if i < 2:
            hm = _mm_layer(hm, p, degp, ws, wn, n2, 1024)
        else:
            out = _mm_final(hm, p, degp, ws, wn,
                            W_out, b_out.reshape(1, F), n2, 1024)
    return out[:n]
